# baseline (speedup 1.0000x reference)
import functools

import numpy as np
import jax
import jax.numpy as jnp
from jax.experimental import pallas as pl
from jax.experimental.pallas import tpu as pltpu

_SIZES = [48, 24, 12, 6, 3]
_STRIDES = [8, 16, 32, 64, 128]
_C = 192
_NCLS = 80
_B = 2
_EPS = 1e-5
_M = 128


def _mm(a, b):
    return jnp.dot(a, b, preferred_element_type=jnp.float32)


def _level_body(Wp, E, CB, T, stride,
                x_ref, w1_ref, a1_ref, w2c_ref, a2c_ref, w2r_ref, a2r_ref,
                wfc_ref, wfr_ref, bf_ref, mask_ref, smask_ref,
                out_ref, h1_ref):
    b = pl.program_id(0)
    i = pl.program_id(1)

    @pl.when(jnp.logical_and(b == 0, i == 0))
    def _init():
        z = jnp.zeros((2 * _C, _M), jnp.bfloat16)
        h1_ref[:, pl.ds(0, _M)] = z
        h1_ref[:, pl.ds(_M + T * CB, _M)] = z

    @pl.when(i < T)
    def _conv1():
        xw = x_ref[pl.ds(b, 1), :, pl.ds(i * CB, CB + 2 * _M)][0]
        acc = None
        for dy in range(3):
            for dx in range(3):
                off = (_M - E) + dy * Wp + dx
                t = _mm(w1_ref[3 * dy + dx], xw[:, off:off + CB])
                acc = t if acc is None else acc + t
        h1 = jnp.maximum(acc * a1_ref[:, 0:1] + a1_ref[:, 1:2], 0.0)
        h1 = (h1 * mask_ref[0:1]).astype(jnp.bfloat16)
        h1_ref[:, pl.ds(_M + i * CB, CB)] = h1

    @pl.when(i >= 1)
    def _conv2():
        j = i - 1
        hw = h1_ref[:, pl.ds(j * CB, CB + 2 * _M)]
        acc_c = None
        acc_r = None
        for dy in range(3):
            for dx in range(3):
                off = (_M - E) + dy * Wp + dx
                hs = hw[:, off:off + CB]
                tc = _mm(w2c_ref[3 * dy + dx], hs[0:_C])
                tr = _mm(w2r_ref[3 * dy + dx], hs[_C:2 * _C])
                acc_c = tc if acc_c is None else acc_c + tc
                acc_r = tr if acc_r is None else acc_r + tr
        h2c = jnp.maximum(acc_c * a2c_ref[:, 0:1] + a2c_ref[:, 1:2], 0.0)
        h2r = jnp.maximum(acc_r * a2r_ref[:, 0:1] + a2r_ref[:, 1:2], 0.0)
        yf = (_mm(wfc_ref[...], h2c.astype(jnp.bfloat16))
              + _mm(wfr_ref[...], h2r.astype(jnp.bfloat16))
              + bf_ref[:, :])
        out_ref[0] = jnp.where(smask_ref[:, :] > 0.0,
                               jnp.maximum(yf * stride, 0.0), yf)


def _bn_affine(b, g, be, mu, va):
    scale = g / jnp.sqrt(va + _EPS)
    shift = (b - mu) * scale + be
    return jnp.stack([scale, shift], axis=1)


def _taps(w):
    t = jnp.transpose(w, (2, 3, 0, 1)).reshape(9, w.shape[0], w.shape[1])
    return t.astype(jnp.bfloat16)


def _full_spec(shape):
    nd = len(shape)
    return pl.BlockSpec(shape, lambda b, i: (0,) * nd)


def kernel(fpn0, fpn1, fpn2, fpn3, fpn4,
           cls_convs_w, cls_convs_b, cls_bn_gamma, cls_bn_beta, cls_bn_mean,
           cls_bn_var, cls_final_w, cls_final_b,
           reg_convs_w, reg_convs_b, reg_bn_gamma, reg_bn_beta, reg_bn_mean,
           reg_bn_var, reg_final_w, reg_final_b):
    fpns = (fpn0, fpn1, fpn2, fpn3, fpn4)

    w1 = _taps(jnp.concatenate([cls_convs_w[0], reg_convs_w[0]], axis=0))
    a1 = jnp.concatenate([
        _bn_affine(cls_convs_b[0], cls_bn_gamma[0], cls_bn_beta[0],
                   cls_bn_mean[0], cls_bn_var[0]),
        _bn_affine(reg_convs_b[0], reg_bn_gamma[0], reg_bn_beta[0],
                   reg_bn_mean[0], reg_bn_var[0])], axis=0)
    w2c_t = _taps(cls_convs_w[1])
    w2r_t = _taps(reg_convs_w[1])
    a2c = _bn_affine(cls_convs_b[1], cls_bn_gamma[1], cls_bn_beta[1],
                     cls_bn_mean[1], cls_bn_var[1])
    a2r = _bn_affine(reg_convs_b[1], reg_bn_gamma[1], reg_bn_beta[1],
                     reg_bn_mean[1], reg_bn_var[1])

    wfc = jnp.concatenate([cls_final_w[:, :, 0, 0],
                           jnp.zeros((5, _C), jnp.float32)],
                          axis=0).astype(jnp.bfloat16)
    wfr = jnp.concatenate([jnp.zeros((_NCLS, _C), jnp.float32),
                           reg_final_w[:, :, 0, 0]],
                          axis=0).astype(jnp.bfloat16)
    bf = jnp.concatenate([cls_final_b, reg_final_b])[:, None]
    smask = jnp.asarray(
        np.concatenate([np.zeros(81, np.float32),
                        np.ones(4, np.float32)])[:, None])

    outs_cls, outs_reg, outs_cent = [], [], []
    for lvl, (x, H, stride) in enumerate(zip(fpns, _SIZES, _STRIDES)):
        Hp = H + 2
        Wp = H + 2
        P = Hp * Wp
        E = Wp + 1
        CB = min(512, -(-P // 128) * 128)
        T = -(-P // CB)
        Ppad = T * CB

        xpad = jnp.pad(x, ((0, 0), (0, 0), (1, 1), (1, 1)))
        x_flat = xpad.reshape(_B, _C, P)
        x_full = jnp.pad(x_flat, ((0, 0), (0, 0), (_M, _M + Ppad - P))
                         ).astype(jnp.bfloat16)

        m = np.zeros((Hp, Wp), np.float32)
        m[1:H + 1, 1:H + 1] = 1.0
        mask = jnp.asarray(np.pad(m.reshape(1, P), ((0, 0), (0, Ppad - P))))

        body = functools.partial(_level_body, Wp, E, CB, T, float(stride))
        out = pl.pallas_call(
            body,
            grid=(_B, T + 1),
            in_specs=[
                _full_spec(x_full.shape),
                _full_spec(w1.shape),
                _full_spec(a1.shape),
                _full_spec(w2c_t.shape),
                _full_spec(a2c.shape),
                _full_spec(w2r_t.shape),
                _full_spec(a2r.shape),
                _full_spec(wfc.shape),
                _full_spec(wfr.shape),
                _full_spec(bf.shape),
                pl.BlockSpec((1, CB),
                             lambda b, i, _T=T: (0, jnp.minimum(i, _T - 1))),
                _full_spec(smask.shape),
            ],
            out_specs=pl.BlockSpec((1, 85, CB),
                                   lambda b, i: (b, 0, jnp.maximum(i - 1, 0))),
            out_shape=jax.ShapeDtypeStruct((_B, 85, Ppad), jnp.float32),
            scratch_shapes=[
                pltpu.VMEM((2 * _C, Ppad + 2 * _M), jnp.bfloat16),
            ],
        )(x_full, w1, a1, w2c_t, a2c, w2r_t, a2r, wfc, wfr, bf,
          mask, smask)

        o = out[:, :, :P].reshape(_B, 85, Hp, Wp)[:, :, 1:H + 1, 1:H + 1]
        outs_cls.append(o[:, 0:_NCLS])
        outs_cent.append(o[:, _NCLS:_NCLS + 1])
        outs_reg.append(o[:, _NCLS + 1:_NCLS + 5])

    return tuple(outs_cls) + tuple(outs_reg) + tuple(outs_cent)


# speedup vs baseline: 1.0944x; 1.0944x over previous
import functools

import numpy as np
import jax
import jax.numpy as jnp
from jax.experimental import pallas as pl
from jax.experimental.pallas import tpu as pltpu

_SIZES = [48, 24, 12, 6, 3]
_STRIDES = [8, 16, 32, 64, 128]
_C = 192
_NCLS = 80
_B = 2
_EPS = 1e-5
_M = 128


def _mm(a, b):
    return jnp.dot(a, b, preferred_element_type=jnp.float32)


def _level_body(Wp, E, CB, T, stride,
                x_ref, w1_ref, a1_ref, w2c_ref, a2c_ref, w2r_ref, a2r_ref,
                wfc_ref, wfr_ref, bf_ref, mask_ref, smask_ref,
                out_ref, h1_ref, xcat_ref, hc_ref, hr_ref):
    i = pl.program_id(0)

    @pl.when(i == 0)
    def _init():
        z = jnp.zeros((2 * _C, _M), jnp.bfloat16)
        h1_ref[:, pl.ds(0, _M)] = z
        h1_ref[:, pl.ds(_M + T * CB, _M)] = z

    @pl.when(i < T)
    def _conv1():
        xw = x_ref[:, pl.ds(i * CB, CB + 2 * _M)]
        for k in range(9):
            off = (_M - E) + (k // 3) * Wp + (k % 3)
            xcat_ref[k * _C:(k + 1) * _C, :] = xw[:, off:off + CB]
        acc = _mm(w1_ref[...], xcat_ref[...])
        h1 = jnp.maximum(acc * a1_ref[:, 0:1] + a1_ref[:, 1:2], 0.0)
        h1 = (h1 * mask_ref[0:1]).astype(jnp.bfloat16)
        h1_ref[:, pl.ds(_M + i * CB, CB)] = h1

    @pl.when(i >= 1)
    def _conv2():
        j = i - 1
        hw = h1_ref[:, pl.ds(j * CB, CB + 2 * _M)]
        for k in range(9):
            off = (_M - E) + (k // 3) * Wp + (k % 3)
            hs = hw[:, off:off + CB]
            hc_ref[k * _C:(k + 1) * _C, :] = hs[0:_C]
            hr_ref[k * _C:(k + 1) * _C, :] = hs[_C:2 * _C]
        acc_c = _mm(w2c_ref[...], hc_ref[...])
        acc_r = _mm(w2r_ref[...], hr_ref[...])
        h2c = jnp.maximum(acc_c * a2c_ref[:, 0:1] + a2c_ref[:, 1:2], 0.0)
        h2r = jnp.maximum(acc_r * a2r_ref[:, 0:1] + a2r_ref[:, 1:2], 0.0)
        yf = (_mm(wfc_ref[...], h2c.astype(jnp.bfloat16))
              + _mm(wfr_ref[...], h2r.astype(jnp.bfloat16))
              + bf_ref[:, :])
        out_ref[...] = jnp.where(smask_ref[:, :] > 0.0,
                                 jnp.maximum(yf * stride, 0.0), yf)


def _bn_affine(b, g, be, mu, va):
    scale = g / jnp.sqrt(va + _EPS)
    shift = (b - mu) * scale + be
    return jnp.stack([scale, shift], axis=1)


def _taps(w):
    t = jnp.transpose(w, (0, 2, 3, 1)).reshape(w.shape[0], 9 * w.shape[1])
    return t.astype(jnp.bfloat16)


def _full_spec(shape):
    nd = len(shape)
    return pl.BlockSpec(shape, lambda i: (0,) * nd)


def kernel(fpn0, fpn1, fpn2, fpn3, fpn4,
           cls_convs_w, cls_convs_b, cls_bn_gamma, cls_bn_beta, cls_bn_mean,
           cls_bn_var, cls_final_w, cls_final_b,
           reg_convs_w, reg_convs_b, reg_bn_gamma, reg_bn_beta, reg_bn_mean,
           reg_bn_var, reg_final_w, reg_final_b):
    fpns = (fpn0, fpn1, fpn2, fpn3, fpn4)

    w1 = _taps(jnp.concatenate([cls_convs_w[0], reg_convs_w[0]], axis=0))
    a1 = jnp.concatenate([
        _bn_affine(cls_convs_b[0], cls_bn_gamma[0], cls_bn_beta[0],
                   cls_bn_mean[0], cls_bn_var[0]),
        _bn_affine(reg_convs_b[0], reg_bn_gamma[0], reg_bn_beta[0],
                   reg_bn_mean[0], reg_bn_var[0])], axis=0)
    w2c_t = _taps(cls_convs_w[1])
    w2r_t = _taps(reg_convs_w[1])
    a2c = _bn_affine(cls_convs_b[1], cls_bn_gamma[1], cls_bn_beta[1],
                     cls_bn_mean[1], cls_bn_var[1])
    a2r = _bn_affine(reg_convs_b[1], reg_bn_gamma[1], reg_bn_beta[1],
                     reg_bn_mean[1], reg_bn_var[1])

    wfc = jnp.concatenate([cls_final_w[:, :, 0, 0],
                           jnp.zeros((5, _C), jnp.float32)],
                          axis=0).astype(jnp.bfloat16)
    wfr = jnp.concatenate([jnp.zeros((_NCLS, _C), jnp.float32),
                           reg_final_w[:, :, 0, 0]],
                          axis=0).astype(jnp.bfloat16)
    bf = jnp.concatenate([cls_final_b, reg_final_b])[:, None]
    smask = jnp.asarray(
        np.concatenate([np.zeros(81, np.float32),
                        np.ones(4, np.float32)])[:, None])

    outs_cls, outs_reg, outs_cent = [], [], []
    for lvl, (x, H, stride) in enumerate(zip(fpns, _SIZES, _STRIDES)):
        Hp = H + 2
        Wp = H + 2
        P = _B * Hp * Wp
        E = Wp + 1
        CB = min(512, -(-P // 128) * 128)
        T = -(-P // CB)
        Ppad = T * CB

        xpad = jnp.pad(x, ((0, 0), (0, 0), (1, 1), (1, 1)))
        x_cat = jnp.transpose(xpad, (1, 0, 2, 3)).reshape(_C, P)
        x_full = jnp.pad(x_cat, ((0, 0), (_M, _M + Ppad - P))
                         ).astype(jnp.bfloat16)

        m = np.zeros((_B, Hp, Wp), np.float32)
        m[:, 1:H + 1, 1:H + 1] = 1.0
        m = np.pad(m.reshape(1, P), ((0, 0), (0, Ppad - P)))
        mask = jnp.asarray(m)

        body = functools.partial(_level_body, Wp, E, CB, T, float(stride))
        out = pl.pallas_call(
            body,
            grid=(T + 1,),
            in_specs=[
                _full_spec(x_full.shape),
                _full_spec(w1.shape),
                _full_spec(a1.shape),
                _full_spec(w2c_t.shape),
                _full_spec(a2c.shape),
                _full_spec(w2r_t.shape),
                _full_spec(a2r.shape),
                _full_spec(wfc.shape),
                _full_spec(wfr.shape),
                _full_spec(bf.shape),
                pl.BlockSpec((1, CB),
                             lambda i, _T=T: (0, jnp.minimum(i, _T - 1))),
                _full_spec(smask.shape),
            ],
            out_specs=pl.BlockSpec((85, CB),
                                   lambda i: (0, jnp.maximum(i - 1, 0))),
            out_shape=jax.ShapeDtypeStruct((85, Ppad), jnp.float32),
            scratch_shapes=[
                pltpu.VMEM((2 * _C, Ppad + 2 * _M), jnp.bfloat16),
                pltpu.VMEM((9 * _C, CB), jnp.bfloat16),
                pltpu.VMEM((9 * _C, CB), jnp.bfloat16),
                pltpu.VMEM((9 * _C, CB), jnp.bfloat16),
            ],
        )(x_full, w1, a1, w2c_t, a2c, w2r_t, a2r, wfc, wfr, bf,
          mask, smask)

        o = jnp.transpose(out[:, :P].reshape(85, _B, Hp, Wp), (1, 0, 2, 3))
        o = o[:, :, 1:H + 1, 1:H + 1]
        outs_cls.append(o[:, 0:_NCLS])
        outs_cent.append(o[:, _NCLS:_NCLS + 1])
        outs_reg.append(o[:, _NCLS + 1:_NCLS + 5])

    return tuple(outs_cls) + tuple(outs_reg) + tuple(outs_cent)


# speedup vs baseline: 1.2670x; 1.1577x over previous
import functools

import numpy as np
import jax
import jax.numpy as jnp
from jax.experimental import pallas as pl
from jax.experimental.pallas import tpu as pltpu

_SIZES = [48, 24, 12, 6, 3]
_STRIDES = [8, 16, 32, 64, 128]
_C = 192
_NCLS = 80
_B = 2
_EPS = 1e-5
_M = 128


def _mm(a, b):
    return jnp.dot(a, b, preferred_element_type=jnp.float32)


def _level_body(Wp, E, CB, T, stride,
                x_ref, w1_ref, a1_ref, w2c_ref, a2c_ref, w2r_ref, a2r_ref,
                wfc_ref, wfr_ref, bf_ref, mask_ref, smask_ref,
                out_ref, h1_ref):
    i = pl.program_id(0)

    @pl.when(i == 0)
    def _init():
        z = jnp.zeros((2 * _C, _M), jnp.bfloat16)
        h1_ref[:, pl.ds(0, _M)] = z
        h1_ref[:, pl.ds(_M + T * CB, _M)] = z

    @pl.when(i < T)
    def _conv1():
        xw = x_ref[:, pl.ds(i * CB, CB + 2 * _M)]
        acc = None
        for dy in range(3):
            for dx in range(3):
                off = (_M - E) + dy * Wp + dx
                t = _mm(w1_ref[3 * dy + dx], xw[:, off:off + CB])
                acc = t if acc is None else acc + t
        h1 = jnp.maximum(acc * a1_ref[:, 0:1] + a1_ref[:, 1:2], 0.0)
        h1 = (h1 * mask_ref[0:1]).astype(jnp.bfloat16)
        h1_ref[:, pl.ds(_M + i * CB, CB)] = h1

    @pl.when(i >= 1)
    def _conv2():
        j = i - 1
        hw = h1_ref[:, pl.ds(j * CB, CB + 2 * _M)]
        acc_c = None
        acc_r = None
        for dy in range(3):
            for dx in range(3):
                off = (_M - E) + dy * Wp + dx
                hs = hw[:, off:off + CB]
                tc = _mm(w2c_ref[3 * dy + dx], hs[0:_C])
                tr = _mm(w2r_ref[3 * dy + dx], hs[_C:2 * _C])
                acc_c = tc if acc_c is None else acc_c + tc
                acc_r = tr if acc_r is None else acc_r + tr
        h2c = jnp.maximum(acc_c * a2c_ref[:, 0:1] + a2c_ref[:, 1:2], 0.0)
        h2r = jnp.maximum(acc_r * a2r_ref[:, 0:1] + a2r_ref[:, 1:2], 0.0)
        yf = (_mm(wfc_ref[...], h2c.astype(jnp.bfloat16))
              + _mm(wfr_ref[...], h2r.astype(jnp.bfloat16))
              + bf_ref[:, :])
        out_ref[...] = jnp.where(smask_ref[:, :] > 0.0,
                                 jnp.maximum(yf * stride, 0.0), yf)


def _bn_affine(b, g, be, mu, va):
    scale = g / jnp.sqrt(va + _EPS)
    shift = (b - mu) * scale + be
    return jnp.stack([scale, shift], axis=1)


def _taps(w):
    t = jnp.transpose(w, (2, 3, 0, 1)).reshape(9, w.shape[0], w.shape[1])
    return t.astype(jnp.bfloat16)


def _full_spec(shape):
    nd = len(shape)
    return pl.BlockSpec(shape, lambda i: (0,) * nd)


def kernel(fpn0, fpn1, fpn2, fpn3, fpn4,
           cls_convs_w, cls_convs_b, cls_bn_gamma, cls_bn_beta, cls_bn_mean,
           cls_bn_var, cls_final_w, cls_final_b,
           reg_convs_w, reg_convs_b, reg_bn_gamma, reg_bn_beta, reg_bn_mean,
           reg_bn_var, reg_final_w, reg_final_b):
    fpns = (fpn0, fpn1, fpn2, fpn3, fpn4)

    w1 = _taps(jnp.concatenate([cls_convs_w[0], reg_convs_w[0]], axis=0))
    a1 = jnp.concatenate([
        _bn_affine(cls_convs_b[0], cls_bn_gamma[0], cls_bn_beta[0],
                   cls_bn_mean[0], cls_bn_var[0]),
        _bn_affine(reg_convs_b[0], reg_bn_gamma[0], reg_bn_beta[0],
                   reg_bn_mean[0], reg_bn_var[0])], axis=0)
    w2c_t = _taps(cls_convs_w[1])
    w2r_t = _taps(reg_convs_w[1])
    a2c = _bn_affine(cls_convs_b[1], cls_bn_gamma[1], cls_bn_beta[1],
                     cls_bn_mean[1], cls_bn_var[1])
    a2r = _bn_affine(reg_convs_b[1], reg_bn_gamma[1], reg_bn_beta[1],
                     reg_bn_mean[1], reg_bn_var[1])

    wfc = jnp.concatenate([cls_final_w[:, :, 0, 0],
                           jnp.zeros((5, _C), jnp.float32)],
                          axis=0).astype(jnp.bfloat16)
    wfr = jnp.concatenate([jnp.zeros((_NCLS, _C), jnp.float32),
                           reg_final_w[:, :, 0, 0]],
                          axis=0).astype(jnp.bfloat16)
    bf = jnp.concatenate([cls_final_b, reg_final_b])[:, None]
    smask = jnp.asarray(
        np.concatenate([np.zeros(81, np.float32),
                        np.ones(4, np.float32)])[:, None])

    outs_cls, outs_reg, outs_cent = [], [], []
    for lvl, (x, H, stride) in enumerate(zip(fpns, _SIZES, _STRIDES)):
        Hp = H + 2
        Wp = H + 2
        P = _B * Hp * Wp
        E = Wp + 1
        CB = min(1024 if P >= 4096 else 512, -(-P // 128) * 128)
        T = -(-P // CB)
        Ppad = T * CB

        xpad = jnp.pad(x, ((0, 0), (0, 0), (1, 1), (1, 1)))
        x_cat = jnp.transpose(xpad, (1, 0, 2, 3)).reshape(_C, P)
        x_full = jnp.pad(x_cat, ((0, 0), (_M, _M + Ppad - P))
                         ).astype(jnp.bfloat16)

        m = np.zeros((_B, Hp, Wp), np.float32)
        m[:, 1:H + 1, 1:H + 1] = 1.0
        m = np.pad(m.reshape(1, P), ((0, 0), (0, Ppad - P)))
        mask = jnp.asarray(m)

        body = functools.partial(_level_body, Wp, E, CB, T, float(stride))
        out = pl.pallas_call(
            body,
            grid=(T + 1,),
            in_specs=[
                _full_spec(x_full.shape),
                _full_spec(w1.shape),
                _full_spec(a1.shape),
                _full_spec(w2c_t.shape),
                _full_spec(a2c.shape),
                _full_spec(w2r_t.shape),
                _full_spec(a2r.shape),
                _full_spec(wfc.shape),
                _full_spec(wfr.shape),
                _full_spec(bf.shape),
                pl.BlockSpec((1, CB),
                             lambda i, _T=T: (0, jnp.minimum(i, _T - 1))),
                _full_spec(smask.shape),
            ],
            out_specs=pl.BlockSpec((85, CB),
                                   lambda i: (0, jnp.maximum(i - 1, 0))),
            out_shape=jax.ShapeDtypeStruct((85, Ppad), jnp.float32),
            scratch_shapes=[
                pltpu.VMEM((2 * _C, Ppad + 2 * _M), jnp.bfloat16),
            ],
        )(x_full, w1, a1, w2c_t, a2c, w2r_t, a2r, wfc, wfr, bf,
          mask, smask)

        o = jnp.transpose(out[:, :P].reshape(85, _B, Hp, Wp), (1, 0, 2, 3))
        o = o[:, :, 1:H + 1, 1:H + 1]
        outs_cls.append(o[:, 0:_NCLS])
        outs_cent.append(o[:, _NCLS:_NCLS + 1])
        outs_reg.append(o[:, _NCLS + 1:_NCLS + 5])

    return tuple(outs_cls) + tuple(outs_reg) + tuple(outs_cent)


# speedup vs baseline: 1.2790x; 1.0094x over previous
import functools

import numpy as np
import jax
import jax.numpy as jnp
from jax.experimental import pallas as pl
from jax.experimental.pallas import tpu as pltpu

_SIZES = [48, 24, 12, 6, 3]
_STRIDES = [8, 16, 32, 64, 128]
_C = 192
_NCLS = 80
_B = 2
_EPS = 1e-5
_M = 128


def _mm(a, b):
    return jnp.dot(a, b, preferred_element_type=jnp.float32)


def _level_body(Wp, E, CB, T, stride,
                x_ref, w1_ref, a1_ref, w2c_ref, a2c_ref, w2r_ref, a2r_ref,
                wfc_ref, wfr_ref, bf_ref, mask_ref, smask_ref,
                out_ref, h1_ref):
    i = pl.program_id(0)

    @pl.when(i == 0)
    def _init():
        z = jnp.zeros((2 * _C, _M), jnp.bfloat16)
        h1_ref[:, pl.ds(0, _M)] = z
        h1_ref[:, pl.ds(_M + T * CB, _M)] = z

    @pl.when(i < T)
    def _conv1():
        xw = x_ref[:, pl.ds(i * CB, CB + 2 * _M)]
        acc = None
        for dy in range(3):
            for dx in range(3):
                off = (_M - E) + dy * Wp + dx
                t = _mm(w1_ref[3 * dy + dx], xw[:, off:off + CB])
                acc = t if acc is None else acc + t
        h1 = jnp.maximum(acc * a1_ref[:, 0:1] + a1_ref[:, 1:2], 0.0)
        h1 = (h1 * mask_ref[0:1]).astype(jnp.bfloat16)
        h1_ref[:, pl.ds(_M + i * CB, CB)] = h1

    @pl.when(i >= 1)
    def _conv2():
        j = i - 1
        hw = h1_ref[:, pl.ds(j * CB, CB + 2 * _M)]
        acc_c = None
        acc_r = None
        for dy in range(3):
            for dx in range(3):
                off = (_M - E) + dy * Wp + dx
                hs = hw[:, off:off + CB]
                tc = _mm(w2c_ref[3 * dy + dx], hs[0:_C])
                tr = _mm(w2r_ref[3 * dy + dx], hs[_C:2 * _C])
                acc_c = tc if acc_c is None else acc_c + tc
                acc_r = tr if acc_r is None else acc_r + tr
        h2c = jnp.maximum(acc_c * a2c_ref[:, 0:1] + a2c_ref[:, 1:2], 0.0)
        h2r = jnp.maximum(acc_r * a2r_ref[:, 0:1] + a2r_ref[:, 1:2], 0.0)
        yf = (_mm(wfc_ref[...], h2c.astype(jnp.bfloat16))
              + _mm(wfr_ref[...], h2r.astype(jnp.bfloat16))
              + bf_ref[:, :])
        out_ref[...] = jnp.where(smask_ref[:, :] > 0.0,
                                 jnp.maximum(yf * stride, 0.0), yf)


def _bn_affine(b, g, be, mu, va):
    scale = g / jnp.sqrt(va + _EPS)
    shift = (b - mu) * scale + be
    return jnp.stack([scale, shift], axis=1)


def _taps(w):
    t = jnp.transpose(w, (2, 3, 0, 1)).reshape(9, w.shape[0], w.shape[1])
    return t.astype(jnp.bfloat16)


def _full_spec(shape):
    nd = len(shape)
    return pl.BlockSpec(shape, lambda i: (0,) * nd)


def kernel(fpn0, fpn1, fpn2, fpn3, fpn4,
           cls_convs_w, cls_convs_b, cls_bn_gamma, cls_bn_beta, cls_bn_mean,
           cls_bn_var, cls_final_w, cls_final_b,
           reg_convs_w, reg_convs_b, reg_bn_gamma, reg_bn_beta, reg_bn_mean,
           reg_bn_var, reg_final_w, reg_final_b):
    fpns = (fpn0, fpn1, fpn2, fpn3, fpn4)

    w1 = _taps(jnp.concatenate([cls_convs_w[0], reg_convs_w[0]], axis=0))
    a1 = jnp.concatenate([
        _bn_affine(cls_convs_b[0], cls_bn_gamma[0], cls_bn_beta[0],
                   cls_bn_mean[0], cls_bn_var[0]),
        _bn_affine(reg_convs_b[0], reg_bn_gamma[0], reg_bn_beta[0],
                   reg_bn_mean[0], reg_bn_var[0])], axis=0)
    w2c_t = _taps(cls_convs_w[1])
    w2r_t = _taps(reg_convs_w[1])
    a2c = _bn_affine(cls_convs_b[1], cls_bn_gamma[1], cls_bn_beta[1],
                     cls_bn_mean[1], cls_bn_var[1])
    a2r = _bn_affine(reg_convs_b[1], reg_bn_gamma[1], reg_bn_beta[1],
                     reg_bn_mean[1], reg_bn_var[1])

    wfc = jnp.concatenate([cls_final_w[:, :, 0, 0],
                           jnp.zeros((5, _C), jnp.float32)],
                          axis=0).astype(jnp.bfloat16)
    wfr = jnp.concatenate([jnp.zeros((_NCLS, _C), jnp.float32),
                           reg_final_w[:, :, 0, 0]],
                          axis=0).astype(jnp.bfloat16)
    bf = jnp.concatenate([cls_final_b, reg_final_b])[:, None]
    smask = jnp.asarray(
        np.concatenate([np.zeros(81, np.float32),
                        np.ones(4, np.float32)])[:, None])

    outs_cls, outs_reg, outs_cent = [], [], []
    for lvl, (x, H, stride) in enumerate(zip(fpns, _SIZES, _STRIDES)):
        Hp = H + 2
        Wp = H + 2
        P = _B * Hp * Wp
        E = Wp + 1
        CB = min(1280 if P >= 4096 else 512, -(-P // 128) * 128)
        T = -(-P // CB)
        Ppad = T * CB

        xpad = jnp.pad(x, ((0, 0), (0, 0), (1, 1), (1, 1)))
        x_cat = jnp.transpose(xpad, (1, 0, 2, 3)).reshape(_C, P)
        x_full = jnp.pad(x_cat, ((0, 0), (_M, _M + Ppad - P))
                         ).astype(jnp.bfloat16)

        m = np.zeros((_B, Hp, Wp), np.float32)
        m[:, 1:H + 1, 1:H + 1] = 1.0
        m = np.pad(m.reshape(1, P), ((0, 0), (0, Ppad - P)))
        mask = jnp.asarray(m)

        body = functools.partial(_level_body, Wp, E, CB, T, float(stride))
        out = pl.pallas_call(
            body,
            grid=(T + 1,),
            in_specs=[
                _full_spec(x_full.shape),
                _full_spec(w1.shape),
                _full_spec(a1.shape),
                _full_spec(w2c_t.shape),
                _full_spec(a2c.shape),
                _full_spec(w2r_t.shape),
                _full_spec(a2r.shape),
                _full_spec(wfc.shape),
                _full_spec(wfr.shape),
                _full_spec(bf.shape),
                pl.BlockSpec((1, CB),
                             lambda i, _T=T: (0, jnp.minimum(i, _T - 1))),
                _full_spec(smask.shape),
            ],
            out_specs=pl.BlockSpec((85, CB),
                                   lambda i: (0, jnp.maximum(i - 1, 0))),
            out_shape=jax.ShapeDtypeStruct((85, Ppad), jnp.float32),
            scratch_shapes=[
                pltpu.VMEM((2 * _C, Ppad + 2 * _M), jnp.bfloat16),
            ],
        )(x_full, w1, a1, w2c_t, a2c, w2r_t, a2r, wfc, wfr, bf,
          mask, smask)

        o = jnp.transpose(out[:, :P].reshape(85, _B, Hp, Wp), (1, 0, 2, 3))
        o = o[:, :, 1:H + 1, 1:H + 1]
        outs_cls.append(o[:, 0:_NCLS])
        outs_cent.append(o[:, _NCLS:_NCLS + 1])
        outs_reg.append(o[:, _NCLS + 1:_NCLS + 5])

    return tuple(outs_cls) + tuple(outs_reg) + tuple(outs_cent)


# speedup vs baseline: 1.2886x; 1.0076x over previous
import functools

import numpy as np
import jax
import jax.numpy as jnp
from jax.experimental import pallas as pl
from jax.experimental.pallas import tpu as pltpu

_SIZES = [48, 24, 12, 6, 3]
_STRIDES = [8, 16, 32, 64, 128]
_C = 192
_NCLS = 80
_B = 2
_EPS = 1e-5
_M = 128


def _mm(a, b):
    return jnp.dot(a, b, preferred_element_type=jnp.float32)


def _level_body(Wp, E, CB, T, stride,
                x_ref, w1_ref, a1_ref, w2c_ref, a2c_ref, w2r_ref, a2r_ref,
                wfc_ref, wfr_ref, bf_ref, mask_ref, smask_ref,
                out_ref, h1_ref):
    i = pl.program_id(0)

    @pl.when(i == 0)
    def _init():
        z = jnp.zeros((2 * _C, _M), jnp.bfloat16)
        h1_ref[:, pl.ds(0, _M)] = z
        h1_ref[:, pl.ds(_M + T * CB, _M)] = z

    @pl.when(i < T)
    def _conv1():
        xw = x_ref[:, pl.ds(i * CB, CB + 2 * _M)]
        acc = None
        for dy in range(3):
            for dx in range(3):
                off = (_M - E) + dy * Wp + dx
                t = _mm(w1_ref[3 * dy + dx], xw[:, off:off + CB])
                acc = t if acc is None else acc + t
        h1 = jnp.maximum(acc * a1_ref[:, 0:1] + a1_ref[:, 1:2], 0.0)
        h1 = (h1 * mask_ref[0:1]).astype(jnp.bfloat16)
        h1_ref[:, pl.ds(_M + i * CB, CB)] = h1

    @pl.when(i >= 1)
    def _conv2():
        j = i - 1
        hw = h1_ref[:, pl.ds(j * CB, CB + 2 * _M)]
        acc_c = None
        acc_r = None
        for dy in range(3):
            for dx in range(3):
                off = (_M - E) + dy * Wp + dx
                hs = hw[:, off:off + CB]
                tc = _mm(w2c_ref[3 * dy + dx], hs[0:_C])
                tr = _mm(w2r_ref[3 * dy + dx], hs[_C:2 * _C])
                acc_c = tc if acc_c is None else acc_c + tc
                acc_r = tr if acc_r is None else acc_r + tr
        h2c = jnp.maximum(acc_c * a2c_ref[:, 0:1] + a2c_ref[:, 1:2], 0.0)
        h2r = jnp.maximum(acc_r * a2r_ref[:, 0:1] + a2r_ref[:, 1:2], 0.0)
        yf = (_mm(wfc_ref[...], h2c.astype(jnp.bfloat16))
              + _mm(wfr_ref[...], h2r.astype(jnp.bfloat16))
              + bf_ref[:, :])
        out_ref[...] = jnp.where(smask_ref[:, :] > 0.0,
                                 jnp.maximum(yf * stride, 0.0), yf)


def _bn_affine(b, g, be, mu, va):
    scale = g / jnp.sqrt(va + _EPS)
    shift = (b - mu) * scale + be
    return jnp.stack([scale, shift], axis=1)


def _taps(w):
    t = jnp.transpose(w, (2, 3, 0, 1)).reshape(9, w.shape[0], w.shape[1])
    return t.astype(jnp.bfloat16)


def _full_spec(shape):
    nd = len(shape)
    return pl.BlockSpec(shape, lambda i: (0,) * nd)


def kernel(fpn0, fpn1, fpn2, fpn3, fpn4,
           cls_convs_w, cls_convs_b, cls_bn_gamma, cls_bn_beta, cls_bn_mean,
           cls_bn_var, cls_final_w, cls_final_b,
           reg_convs_w, reg_convs_b, reg_bn_gamma, reg_bn_beta, reg_bn_mean,
           reg_bn_var, reg_final_w, reg_final_b):
    fpns = (fpn0, fpn1, fpn2, fpn3, fpn4)

    w1 = _taps(jnp.concatenate([cls_convs_w[0], reg_convs_w[0]], axis=0))
    a1 = jnp.concatenate([
        _bn_affine(cls_convs_b[0], cls_bn_gamma[0], cls_bn_beta[0],
                   cls_bn_mean[0], cls_bn_var[0]),
        _bn_affine(reg_convs_b[0], reg_bn_gamma[0], reg_bn_beta[0],
                   reg_bn_mean[0], reg_bn_var[0])], axis=0)
    w2c_t = _taps(cls_convs_w[1])
    w2r_t = _taps(reg_convs_w[1])
    a2c = _bn_affine(cls_convs_b[1], cls_bn_gamma[1], cls_bn_beta[1],
                     cls_bn_mean[1], cls_bn_var[1])
    a2r = _bn_affine(reg_convs_b[1], reg_bn_gamma[1], reg_bn_beta[1],
                     reg_bn_mean[1], reg_bn_var[1])

    wfc = jnp.concatenate([cls_final_w[:, :, 0, 0],
                           jnp.zeros((5, _C), jnp.float32)],
                          axis=0).astype(jnp.bfloat16)
    wfr = jnp.concatenate([jnp.zeros((_NCLS, _C), jnp.float32),
                           reg_final_w[:, :, 0, 0]],
                          axis=0).astype(jnp.bfloat16)
    bf = jnp.concatenate([cls_final_b, reg_final_b])[:, None]
    smask = jnp.asarray(
        np.concatenate([np.zeros(81, np.float32),
                        np.ones(4, np.float32)])[:, None])

    outs_cls, outs_reg, outs_cent = [], [], []
    for lvl, (x, H, stride) in enumerate(zip(fpns, _SIZES, _STRIDES)):
        Hp = H + 2
        Wp = H + 2
        P = _B * Hp * Wp
        E = Wp + 1
        CB = min(2560 if P >= 4096 else (768 if P >= 1024 else 512), -(-P // 128) * 128)
        T = -(-P // CB)
        Ppad = T * CB

        xpad = jnp.pad(x, ((0, 0), (0, 0), (1, 1), (1, 1)))
        x_cat = jnp.transpose(xpad, (1, 0, 2, 3)).reshape(_C, P)
        x_full = jnp.pad(x_cat, ((0, 0), (_M, _M + Ppad - P))
                         ).astype(jnp.bfloat16)

        m = np.zeros((_B, Hp, Wp), np.float32)
        m[:, 1:H + 1, 1:H + 1] = 1.0
        m = np.pad(m.reshape(1, P), ((0, 0), (0, Ppad - P)))
        mask = jnp.asarray(m)

        body = functools.partial(_level_body, Wp, E, CB, T, float(stride))
        out = pl.pallas_call(
            body,
            grid=(T + 1,),
            in_specs=[
                _full_spec(x_full.shape),
                _full_spec(w1.shape),
                _full_spec(a1.shape),
                _full_spec(w2c_t.shape),
                _full_spec(a2c.shape),
                _full_spec(w2r_t.shape),
                _full_spec(a2r.shape),
                _full_spec(wfc.shape),
                _full_spec(wfr.shape),
                _full_spec(bf.shape),
                pl.BlockSpec((1, CB),
                             lambda i, _T=T: (0, jnp.minimum(i, _T - 1))),
                _full_spec(smask.shape),
            ],
            out_specs=pl.BlockSpec((85, CB),
                                   lambda i: (0, jnp.maximum(i - 1, 0))),
            out_shape=jax.ShapeDtypeStruct((85, Ppad), jnp.float32),
            scratch_shapes=[
                pltpu.VMEM((2 * _C, Ppad + 2 * _M), jnp.bfloat16),
            ],
        )(x_full, w1, a1, w2c_t, a2c, w2r_t, a2r, wfc, wfr, bf,
          mask, smask)

        o = jnp.transpose(out[:, :P].reshape(85, _B, Hp, Wp), (1, 0, 2, 3))
        o = o[:, :, 1:H + 1, 1:H + 1]
        outs_cls.append(o[:, 0:_NCLS])
        outs_cent.append(o[:, _NCLS:_NCLS + 1])
        outs_reg.append(o[:, _NCLS + 1:_NCLS + 5])

    return tuple(outs_cls) + tuple(outs_reg) + tuple(outs_cent)


# speedup vs baseline: 1.4018x; 1.0878x over previous
import functools

import numpy as np
import jax
import jax.numpy as jnp
from jax.experimental import pallas as pl
from jax.experimental.pallas import tpu as pltpu

_SIZES = [48, 24, 12, 6, 3]
_STRIDES = [8, 16, 32, 64, 128]
_C = 192
_NCLS = 80
_B = 2
_EPS = 1e-5
_M = 128


def _mm(a, b):
    return jnp.dot(a, b, preferred_element_type=jnp.float32)


def _level_body(Wp, E, CB, T,
                x_ref, w1_ref, a1_ref, w2c_ref, a2c_ref, w2r_ref, a2r_ref,
                wfc_ref, wfr_ref, bf_ref, mask_ref, svec_ref, smask_ref,
                out_ref, h1_ref):
    i = pl.program_id(0)

    @pl.when(i == 0)
    def _init():
        z = jnp.zeros((2 * _C, _M), jnp.bfloat16)
        h1_ref[:, pl.ds(0, _M)] = z
        h1_ref[:, pl.ds(_M + T * CB, _M)] = z

    @pl.when(i < T)
    def _conv1():
        xw = x_ref[:, pl.ds(i * CB, CB + 2 * _M)]
        acc = None
        for dy in range(3):
            for dx in range(3):
                off = (_M - E) + dy * Wp + dx
                t = _mm(w1_ref[3 * dy + dx], xw[:, off:off + CB])
                acc = t if acc is None else acc + t
        h1 = jnp.maximum(acc * a1_ref[:, 0:1] + a1_ref[:, 1:2], 0.0)
        h1 = (h1 * mask_ref[0:1]).astype(jnp.bfloat16)
        h1_ref[:, pl.ds(_M + i * CB, CB)] = h1

    @pl.when(i >= 1)
    def _conv2():
        j = i - 1
        hw = h1_ref[:, pl.ds(j * CB, CB + 2 * _M)]
        acc_c = None
        acc_r = None
        for dy in range(3):
            for dx in range(3):
                off = (_M - E) + dy * Wp + dx
                hs = hw[:, off:off + CB]
                tc = _mm(w2c_ref[3 * dy + dx], hs[0:_C])
                tr = _mm(w2r_ref[3 * dy + dx], hs[_C:2 * _C])
                acc_c = tc if acc_c is None else acc_c + tc
                acc_r = tr if acc_r is None else acc_r + tr
        h2c = jnp.maximum(acc_c * a2c_ref[:, 0:1] + a2c_ref[:, 1:2], 0.0)
        h2r = jnp.maximum(acc_r * a2r_ref[:, 0:1] + a2r_ref[:, 1:2], 0.0)
        yf = (_mm(wfc_ref[...], h2c.astype(jnp.bfloat16))
              + _mm(wfr_ref[...], h2r.astype(jnp.bfloat16))
              + bf_ref[:, :])
        out_ref[...] = jnp.where(smask_ref[:, :] > 0.0,
                                 jnp.maximum(yf * svec_ref[0:1], 0.0), yf)


def _bn_affine(b, g, be, mu, va):
    scale = g / jnp.sqrt(va + _EPS)
    shift = (b - mu) * scale + be
    return jnp.stack([scale, shift], axis=1)


def _taps(w):
    t = jnp.transpose(w, (2, 3, 0, 1)).reshape(9, w.shape[0], w.shape[1])
    return t.astype(jnp.bfloat16)


def _full_spec(shape):
    nd = len(shape)
    return pl.BlockSpec(shape, lambda i: (0,) * nd)


def kernel(fpn0, fpn1, fpn2, fpn3, fpn4,
           cls_convs_w, cls_convs_b, cls_bn_gamma, cls_bn_beta, cls_bn_mean,
           cls_bn_var, cls_final_w, cls_final_b,
           reg_convs_w, reg_convs_b, reg_bn_gamma, reg_bn_beta, reg_bn_mean,
           reg_bn_var, reg_final_w, reg_final_b):
    fpns = (fpn0, fpn1, fpn2, fpn3, fpn4)

    w1 = _taps(jnp.concatenate([cls_convs_w[0], reg_convs_w[0]], axis=0))
    a1 = jnp.concatenate([
        _bn_affine(cls_convs_b[0], cls_bn_gamma[0], cls_bn_beta[0],
                   cls_bn_mean[0], cls_bn_var[0]),
        _bn_affine(reg_convs_b[0], reg_bn_gamma[0], reg_bn_beta[0],
                   reg_bn_mean[0], reg_bn_var[0])], axis=0)
    w2c_t = _taps(cls_convs_w[1])
    w2r_t = _taps(reg_convs_w[1])
    a2c = _bn_affine(cls_convs_b[1], cls_bn_gamma[1], cls_bn_beta[1],
                     cls_bn_mean[1], cls_bn_var[1])
    a2r = _bn_affine(reg_convs_b[1], reg_bn_gamma[1], reg_bn_beta[1],
                     reg_bn_mean[1], reg_bn_var[1])

    wfc = jnp.concatenate([cls_final_w[:, :, 0, 0],
                           jnp.zeros((5, _C), jnp.float32)],
                          axis=0).astype(jnp.bfloat16)
    wfr = jnp.concatenate([jnp.zeros((_NCLS, _C), jnp.float32),
                           reg_final_w[:, :, 0, 0]],
                          axis=0).astype(jnp.bfloat16)
    bf = jnp.concatenate([cls_final_b, reg_final_b])[:, None]
    smask = jnp.asarray(
        np.concatenate([np.zeros(81, np.float32),
                        np.ones(4, np.float32)])[:, None])

    groups = [[0], [1], [2, 3, 4]]
    outs_cls = [None] * 5
    outs_reg = [None] * 5
    outs_cent = [None] * 5
    for lvls in groups:
        Wp = _SIZES[lvls[0]] + 2
        E = Wp + 1
        segs = []
        x_parts = []
        pos = 0
        m_parts = []
        s_parts = []
        for lvl in lvls:
            H = _SIZES[lvl]
            Hp = H + 2
            seg = _B * Hp * Wp
            xp = jnp.pad(fpns[lvl],
                         ((0, 0), (0, 0), (1, 1), (1, Wp - 1 - H)))
            x_parts.append(jnp.transpose(xp, (1, 0, 2, 3)).reshape(_C, seg))
            m = np.zeros((_B, Hp, Wp), np.float32)
            m[:, 1:H + 1, 1:H + 1] = 1.0
            m_parts.append(m.reshape(-1))
            s_parts.append(np.full(seg, float(_STRIDES[lvl]), np.float32))
            segs.append((lvl, H, Hp, pos, seg))
            pos += seg
        P = pos
        CB = min(2560 if P >= 4096 else (768 if P >= 512 else 512),
                 -(-P // 128) * 128)
        T = -(-P // CB)
        Ppad = T * CB

        x_cat = x_parts[0] if len(x_parts) == 1 else jnp.concatenate(
            x_parts, axis=1)
        x_full = jnp.pad(x_cat, ((0, 0), (_M, _M + Ppad - P))
                         ).astype(jnp.bfloat16)

        mask = jnp.asarray(np.pad(np.concatenate(m_parts),
                                  (0, Ppad - P)).reshape(1, Ppad))
        svec = jnp.asarray(np.pad(np.concatenate(s_parts),
                                  (0, Ppad - P)).reshape(1, Ppad))

        body = functools.partial(_level_body, Wp, E, CB, T)
        tile_spec = pl.BlockSpec(
            (1, CB), lambda i, _T=T: (0, jnp.minimum(i, _T - 1)))
        out = pl.pallas_call(
            body,
            grid=(T + 1,),
            in_specs=[
                _full_spec(x_full.shape),
                _full_spec(w1.shape),
                _full_spec(a1.shape),
                _full_spec(w2c_t.shape),
                _full_spec(a2c.shape),
                _full_spec(w2r_t.shape),
                _full_spec(a2r.shape),
                _full_spec(wfc.shape),
                _full_spec(wfr.shape),
                _full_spec(bf.shape),
                tile_spec,
                pl.BlockSpec((1, CB), lambda i: (0, jnp.maximum(i - 1, 0))),
                _full_spec(smask.shape),
            ],
            out_specs=pl.BlockSpec((85, CB),
                                   lambda i: (0, jnp.maximum(i - 1, 0))),
            out_shape=jax.ShapeDtypeStruct((85, Ppad), jnp.float32),
            scratch_shapes=[
                pltpu.VMEM((2 * _C, Ppad + 2 * _M), jnp.bfloat16),
            ],
        )(x_full, w1, a1, w2c_t, a2c, w2r_t, a2r, wfc, wfr, bf,
          mask, svec, smask)

        for lvl, H, Hp, start, seg in segs:
            o = out[:, start:start + seg].reshape(85, _B, Hp, Wp)
            o = jnp.transpose(o, (1, 0, 2, 3))[:, :, 1:H + 1, 1:H + 1]
            outs_cls[lvl] = o[:, 0:_NCLS]
            outs_cent[lvl] = o[:, _NCLS:_NCLS + 1]
            outs_reg[lvl] = o[:, _NCLS + 1:_NCLS + 5]

    return tuple(outs_cls) + tuple(outs_reg) + tuple(outs_cent)
